# initial kernel scaffold (unmeasured)
import jax
import jax.numpy as jnp
from jax import lax
from jax.experimental import pallas as pl
from jax.experimental.pallas import tpu as pltpu


def kernel(
    x,
):
    def body(*refs):
        pass

    out_shape = jax.ShapeDtypeStruct(..., jnp.float32)
    return pl.pallas_call(body, out_shape=out_shape)(...)


# baseline (device time: 97381 ns/iter reference)
import functools

import jax
import jax.numpy as jnp
from jax import lax
from jax.experimental import pallas as pl
from jax.experimental.pallas import tpu as pltpu

N_DEV = 32
N_STAGES = 5


def kernel(x):
    _, m, n = x.shape

    def body(x_ref, out_ref, recv_buf, send_sems, recv_sems):
        my = lax.axis_index("i")

        barrier_sem = pltpu.get_barrier_semaphore()
        for s in range(N_STAGES):
            partner = jnp.bitwise_xor(my, 1 << s)
            pl.semaphore_signal(
                barrier_sem, inc=1,
                device_id=(partner,), device_id_type=pl.DeviceIdType.MESH,
            )
        pl.semaphore_wait(barrier_sem, N_STAGES)

        out_ref[:, :] = x_ref[0, :, :]

        for s in range(N_STAGES):
            partner = jnp.bitwise_xor(my, 1 << s)
            rdma = pltpu.make_async_remote_copy(
                src_ref=out_ref,
                dst_ref=recv_buf.at[s],
                send_sem=send_sems.at[s],
                recv_sem=recv_sems.at[s],
                device_id=(partner,),
                device_id_type=pl.DeviceIdType.MESH,
            )
            rdma.start()
            rdma.wait()
            out_ref[:, :] = out_ref[:, :] + recv_buf[s, :, :]

        @functools.partial(
            pl.run_scoped, second_barrier=pltpu.SemaphoreType.REGULAR
        )
        def _(second_barrier):
            for s in range(N_STAGES):
                partner = jnp.bitwise_xor(my, 1 << s)
                pl.semaphore_signal(
                    second_barrier, inc=1,
                    device_id=(partner,), device_id_type=pl.DeviceIdType.MESH,
                )
            pl.semaphore_wait(second_barrier, N_STAGES)

    return pl.pallas_call(
        body,
        out_shape=jax.ShapeDtypeStruct((m, n), jnp.float32),
        in_specs=[pl.BlockSpec(memory_space=pltpu.VMEM)],
        out_specs=pl.BlockSpec(memory_space=pltpu.VMEM),
        scratch_shapes=[
            pltpu.VMEM((N_STAGES, m, n), jnp.float32),
            pltpu.SemaphoreType.DMA((N_STAGES,)),
            pltpu.SemaphoreType.DMA((N_STAGES,)),
        ],
        compiler_params=pltpu.CompilerParams(collective_id=0),
    )(x)


# device time: 50169 ns/iter; 1.9411x vs baseline; 1.9411x over previous
import functools

import jax
import jax.numpy as jnp
from jax import lax
from jax.experimental import pallas as pl
from jax.experimental.pallas import tpu as pltpu

N_DEV = 32
M = 512

RS_MASKS = (1, 3, 4, 8, 16)
RS_HALF = (256, 128, 64, 32, 16)
AG_MASKS = tuple(reversed(RS_MASKS))
AG_LEN = tuple(reversed(RS_HALF))
ALL_PARTNER_MASKS = (1, 3, 4, 8, 16)


def kernel(x):
    _, m, n = x.shape

    def body(x_ref, out_ref, *scratch):
        rs_bufs = scratch[0:5]
        ag_bufs = scratch[5:10]
        send_sems, recv_sems = scratch[10], scratch[11]

        p = lax.axis_index("i")
        k = jnp.bitwise_and(p, 7)
        y = jnp.right_shift(k, 1)
        x_bit = jnp.bitwise_and(k + y, 1)
        z = jnp.right_shift(p, 3)
        sels = (
            x_bit,
            jnp.bitwise_and(y, 1),
            jnp.bitwise_and(jnp.right_shift(y, 1), 1),
            jnp.bitwise_and(z, 1),
            jnp.bitwise_and(jnp.right_shift(z, 1), 1),
        )

        barrier_sem = pltpu.get_barrier_semaphore()
        for mask in ALL_PARTNER_MASKS:
            pl.semaphore_signal(
                barrier_sem, inc=1,
                device_id=(jnp.bitwise_xor(p, mask),),
                device_id_type=pl.DeviceIdType.MESH,
            )
        pl.semaphore_wait(barrier_sem, len(ALL_PARTNER_MASKS))

        out_ref[:, :] = x_ref[0, :, :]

        seg_start = jnp.int32(0)
        for i in range(5):
            half = RS_HALF[i]
            sel = sels[i]
            partner = jnp.bitwise_xor(p, RS_MASKS[i])
            keep_start = seg_start + sel * half
            send_start = seg_start + (1 - sel) * half
            rdma = pltpu.make_async_remote_copy(
                src_ref=out_ref.at[pl.ds(send_start, half)],
                dst_ref=rs_bufs[i],
                send_sem=send_sems.at[i],
                recv_sem=recv_sems.at[i],
                device_id=(partner,),
                device_id_type=pl.DeviceIdType.MESH,
            )
            rdma.start()
            rdma.wait()
            out_ref[pl.ds(keep_start, half), :] = (
                out_ref[pl.ds(keep_start, half), :] + rs_bufs[i][:, :]
            )
            seg_start = keep_start

        for j in range(5):
            seg_len = AG_LEN[j]
            sel = sels[4 - j]
            partner = jnp.bitwise_xor(p, AG_MASKS[j])
            partner_start = seg_start + (1 - 2 * sel) * seg_len
            rdma = pltpu.make_async_remote_copy(
                src_ref=out_ref.at[pl.ds(seg_start, seg_len)],
                dst_ref=ag_bufs[j],
                send_sem=send_sems.at[5 + j],
                recv_sem=recv_sems.at[5 + j],
                device_id=(partner,),
                device_id_type=pl.DeviceIdType.MESH,
            )
            rdma.start()
            rdma.wait()
            out_ref[pl.ds(partner_start, seg_len), :] = ag_bufs[j][:, :]
            seg_start = seg_start - sel * seg_len

        @functools.partial(
            pl.run_scoped, second_barrier=pltpu.SemaphoreType.REGULAR
        )
        def _(second_barrier):
            for mask in ALL_PARTNER_MASKS:
                pl.semaphore_signal(
                    second_barrier, inc=1,
                    device_id=(jnp.bitwise_xor(p, mask),),
                    device_id_type=pl.DeviceIdType.MESH,
                )
            pl.semaphore_wait(second_barrier, len(ALL_PARTNER_MASKS))

    return pl.pallas_call(
        body,
        out_shape=jax.ShapeDtypeStruct((m, n), jnp.float32),
        in_specs=[pl.BlockSpec(memory_space=pltpu.VMEM)],
        out_specs=pl.BlockSpec(memory_space=pltpu.VMEM),
        scratch_shapes=[
            *[pltpu.VMEM((h, n), jnp.float32) for h in RS_HALF],
            *[pltpu.VMEM((l, n), jnp.float32) for l in AG_LEN],
            pltpu.SemaphoreType.DMA((10,)),
            pltpu.SemaphoreType.DMA((10,)),
        ],
        compiler_params=pltpu.CompilerParams(collective_id=0),
    )(x)


# device time: 42938 ns/iter; 2.2679x vs baseline; 1.1684x over previous
import functools

import jax
import jax.numpy as jnp
from jax import lax
from jax.experimental import pallas as pl
from jax.experimental.pallas import tpu as pltpu

N_DEV = 32
SEG = 512 // N_DEV


def kernel(x):
    _, m, n = x.shape

    def body(x_ref, out_ref, rs_buf, rs_send, rs_recv, ag_send, ag_recv):
        p = lax.axis_index("i")

        barrier_sem = pltpu.get_barrier_semaphore()
        for o in range(1, N_DEV):
            pl.semaphore_signal(
                barrier_sem, inc=1,
                device_id=(jnp.mod(p + o, N_DEV),),
                device_id_type=pl.DeviceIdType.MESH,
            )
        pl.semaphore_wait(barrier_sem, N_DEV - 1)

        rs_sends = []
        for o in range(1, N_DEV):
            d = jnp.mod(p + o, N_DEV)
            rdma = pltpu.make_async_remote_copy(
                src_ref=x_ref.at[0, pl.ds(d * SEG, SEG), :],
                dst_ref=rs_buf.at[p],
                send_sem=rs_send.at[o],
                recv_sem=rs_recv.at[p],
                device_id=(d,),
                device_id_type=pl.DeviceIdType.MESH,
            )
            rdma.start()
            rs_sends.append(rdma)

        acc = x_ref[0, pl.ds(p * SEG, SEG), :]
        for o in range(1, N_DEV):
            s = jnp.mod(p + o, N_DEV)
            recv = pltpu.make_async_remote_copy(
                src_ref=rs_buf.at[s],
                dst_ref=rs_buf.at[s],
                send_sem=rs_send.at[o],
                recv_sem=rs_recv.at[s],
                device_id=(p,),
                device_id_type=pl.DeviceIdType.MESH,
            )
            recv.wait_recv()
            acc = acc + rs_buf[s]
        out_ref[pl.ds(p * SEG, SEG), :] = acc

        ag_sends = []
        for o in range(1, N_DEV):
            q = jnp.mod(p + o, N_DEV)
            rdma = pltpu.make_async_remote_copy(
                src_ref=out_ref.at[pl.ds(p * SEG, SEG), :],
                dst_ref=out_ref.at[pl.ds(p * SEG, SEG), :],
                send_sem=ag_send.at[o],
                recv_sem=ag_recv.at[p],
                device_id=(q,),
                device_id_type=pl.DeviceIdType.MESH,
            )
            rdma.start()
            ag_sends.append(rdma)

        for rdma in rs_sends:
            rdma.wait_send()

        for o in range(1, N_DEV):
            s = jnp.mod(p + o, N_DEV)
            recv = pltpu.make_async_remote_copy(
                src_ref=out_ref.at[pl.ds(s * SEG, SEG), :],
                dst_ref=out_ref.at[pl.ds(s * SEG, SEG), :],
                send_sem=ag_send.at[o],
                recv_sem=ag_recv.at[s],
                device_id=(p,),
                device_id_type=pl.DeviceIdType.MESH,
            )
            recv.wait_recv()
        for rdma in ag_sends:
            rdma.wait_send()

        @functools.partial(
            pl.run_scoped, second_barrier=pltpu.SemaphoreType.REGULAR
        )
        def _(second_barrier):
            for o in range(1, N_DEV):
                pl.semaphore_signal(
                    second_barrier, inc=1,
                    device_id=(jnp.mod(p + o, N_DEV),),
                    device_id_type=pl.DeviceIdType.MESH,
                )
            pl.semaphore_wait(second_barrier, N_DEV - 1)

    return pl.pallas_call(
        body,
        out_shape=jax.ShapeDtypeStruct((m, n), jnp.float32),
        in_specs=[pl.BlockSpec(memory_space=pltpu.VMEM)],
        out_specs=pl.BlockSpec(memory_space=pltpu.VMEM),
        scratch_shapes=[
            pltpu.VMEM((N_DEV, SEG, n), jnp.float32),
            pltpu.SemaphoreType.DMA((N_DEV,)),
            pltpu.SemaphoreType.DMA((N_DEV,)),
            pltpu.SemaphoreType.DMA((N_DEV,)),
            pltpu.SemaphoreType.DMA((N_DEV,)),
        ],
        compiler_params=pltpu.CompilerParams(collective_id=0),
    )(x)
